# Initial kernel scaffold; baseline (speedup 1.0000x reference)
import jax
import jax.numpy as jnp
from jax.experimental import pallas as pl


def kernel(x, lengths):
    raise NotImplementedError("write your pallas kernel here")


# speedup vs baseline: 3.5133x; 3.5133x over previous
import functools

import jax
import jax.numpy as jnp
from jax import lax
from jax.experimental import pallas as pl
from jax.experimental.pallas import tpu as pltpu
from jax.experimental.pallas import tpu_sc as plsc

_BLANK = 0
_SEQ = 2048
_BATCH = 16
_ALPHA = 1024
_SBLK = 128
_LANES = 16


def _argmax_block(x_ref, o_ref):
    xb = x_ref[...]
    m = jnp.max(xb, axis=2, keepdims=True)
    idx = lax.broadcasted_iota(jnp.int32, xb.shape, 2)
    ml = jnp.min(jnp.where(xb == m, idx, _ALPHA), axis=2)
    o_ref[...] = ml.astype(jnp.int32).T


def _argmax_tc(x):
    seq, batch, alpha = x.shape
    return pl.pallas_call(
        _argmax_block,
        grid=(seq // _SBLK,),
        in_specs=[pl.BlockSpec((_SBLK, batch, alpha), lambda i: (i, 0, 0))],
        out_specs=pl.BlockSpec((batch, _SBLK), lambda i: (0, i)),
        out_shape=jax.ShapeDtypeStruct((batch, seq), jnp.int32),
    )(x)


def _collapse_body(ml_hbm, len_hbm, tok_hbm, lenout_hbm, row_v, out_v, len_v, tmp_v):
    wid = lax.axis_index("s") * 2 + lax.axis_index("c")

    @pl.when(wid < _BATCH)
    def _():
        b = wid
        pltpu.sync_copy(ml_hbm.at[b], row_v)
        pltpu.sync_copy(len_hbm, len_v)
        lanes = lax.iota(jnp.int32, _LANES)
        lenb = plsc.load_gather(len_v, [jnp.full((_LANES,), b, jnp.int32)])
        last = jnp.full((_LANES,), _LANES - 1, jnp.int32)

        def step(c, rt):
            base = c * _LANES
            out_v[pl.ds(base, _LANES)] = jnp.full((_LANES,), -1, jnp.int32)
            v = row_v[pl.ds(base, _LANES)]
            gpos = base + lanes
            prevv = plsc.load_gather(row_v, [jnp.maximum(gpos - 1, 0)])
            prevv = jnp.where(gpos == 0, _BLANK, prevv)
            keep = (v != _BLANK) & ((prevv == _BLANK) | (v != prevv)) & (gpos < lenb)
            cs = plsc.cumsum(keep.astype(jnp.int32))
            pos = rt + cs - 1
            dest = jnp.where(keep, pos, 0)
            plsc.store_scatter(out_v, [dest], v, mask=keep)
            tmp_v[...] = cs
            return rt + plsc.load_gather(tmp_v, [last])

        rt = lax.fori_loop(
            0, _SEQ // _LANES, step, jnp.zeros((_LANES,), jnp.int32)
        )
        pltpu.sync_copy(out_v, tok_hbm.at[b])
        tmp_v[...] = rt
        pltpu.sync_copy(tmp_v, lenout_hbm.at[b])


@functools.cache
def _collapse_sc():
    return pl.kernel(
        _collapse_body,
        out_type=[
            jax.ShapeDtypeStruct((_BATCH, _SEQ), jnp.int32),
            jax.ShapeDtypeStruct((_BATCH, _LANES), jnp.int32),
        ],
        mesh=plsc.VectorSubcoreMesh(core_axis_name="c", subcore_axis_name="s"),
        compiler_params=pltpu.CompilerParams(needs_layout_passes=False),
        scratch_types=[
            pltpu.VMEM((_SEQ,), jnp.int32),
            pltpu.VMEM((_SEQ,), jnp.int32),
            pltpu.VMEM((_LANES,), jnp.int32),
            pltpu.VMEM((_LANES,), jnp.int32),
        ],
    )


@jax.jit
def kernel(x, lengths):
    ml = _argmax_tc(x)
    tok, lenm = _collapse_sc()(ml, lengths)
    return tok, lenm[:, 0]


# speedup vs baseline: 3.7115x; 1.0564x over previous
import functools

import jax
import jax.numpy as jnp
from jax import lax
from jax.experimental import pallas as pl
from jax.experimental.pallas import tpu as pltpu
from jax.experimental.pallas import tpu_sc as plsc

_BLANK = 0
_SEQ = 2048
_BATCH = 16
_ALPHA = 1024
_SBLK = 128
_LANES = 16


def _argmax_block(x_ref, o_ref):
    xb = x_ref[...]
    m = jnp.max(xb, axis=2, keepdims=True)
    idx = lax.broadcasted_iota(jnp.int32, xb.shape, 2).astype(jnp.float32)
    ml = jnp.min(jnp.where(xb == m, idx, float(_ALPHA)), axis=2)
    o_ref[...] = ml.astype(jnp.int32).T


def _argmax_tc(x):
    seq, batch, alpha = x.shape
    return pl.pallas_call(
        _argmax_block,
        grid=(seq // _SBLK,),
        in_specs=[pl.BlockSpec((_SBLK, batch, alpha), lambda i: (i, 0, 0))],
        out_specs=pl.BlockSpec((batch, _SBLK), lambda i: (0, i)),
        out_shape=jax.ShapeDtypeStruct((batch, seq), jnp.int32),
    )(x)


def _collapse_body(ml_hbm, len_hbm, tok_hbm, lenout_hbm, row_v, out_v, len_v, tmp_v):
    wid = lax.axis_index("s") * 2 + lax.axis_index("c")

    @pl.when(wid < _BATCH)
    def _():
        b = wid
        pltpu.sync_copy(ml_hbm.at[b], row_v)
        pltpu.sync_copy(len_hbm, len_v)
        lanes = lax.iota(jnp.int32, _LANES)
        lenb = plsc.load_gather(len_v, [jnp.full((_LANES,), b, jnp.int32)])
        last = jnp.full((_LANES,), _LANES - 1, jnp.int32)

        def step(c, rt):
            base = c * _LANES
            out_v[pl.ds(base, _LANES)] = jnp.full((_LANES,), -1, jnp.int32)
            v = row_v[pl.ds(base, _LANES)]
            gpos = base + lanes
            prevv = plsc.load_gather(row_v, [jnp.maximum(gpos - 1, 0)])
            prevv = jnp.where(gpos == 0, _BLANK, prevv)
            keep = (v != _BLANK) & ((prevv == _BLANK) | (v != prevv)) & (gpos < lenb)
            cs = plsc.cumsum(keep.astype(jnp.int32))
            pos = rt + cs - 1
            dest = jnp.where(keep, pos, 0)
            plsc.store_scatter(out_v, [dest], v, mask=keep)
            tmp_v[...] = cs
            return rt + plsc.load_gather(tmp_v, [last])

        rt = lax.fori_loop(
            0, _SEQ // _LANES, step, jnp.zeros((_LANES,), jnp.int32)
        )
        pltpu.sync_copy(out_v, tok_hbm.at[b])
        tmp_v[...] = rt
        pltpu.sync_copy(tmp_v, lenout_hbm.at[b])


@functools.cache
def _collapse_sc():
    return pl.kernel(
        _collapse_body,
        out_type=[
            jax.ShapeDtypeStruct((_BATCH, _SEQ), jnp.int32),
            jax.ShapeDtypeStruct((_BATCH, _LANES), jnp.int32),
        ],
        mesh=plsc.VectorSubcoreMesh(core_axis_name="c", subcore_axis_name="s"),
        compiler_params=pltpu.CompilerParams(needs_layout_passes=False),
        scratch_types=[
            pltpu.VMEM((_SEQ,), jnp.int32),
            pltpu.VMEM((_SEQ,), jnp.int32),
            pltpu.VMEM((_LANES,), jnp.int32),
            pltpu.VMEM((_LANES,), jnp.int32),
        ],
    )


@jax.jit
def kernel(x, lengths):
    ml = _argmax_tc(x)
    tok, lenm = _collapse_sc()(ml, lengths)
    return tok, lenm[:, 0]
